# baseline (speedup 1.0000x reference)
import functools

import jax
import jax.numpy as jnp
from jax import lax
from jax.experimental import pallas as pl
from jax.experimental.pallas import tpu as pltpu
from jax.experimental.pallas import tpu_sc as plsc

_B, _A, _M, _T, _C = 1024, 300, 6, 12, 10
_T2 = 2 * _T

_AGENT_THRESH = 0.5
_X_TH = 1.5
_Y_TH = 3.0
_DIS_TH_SQ = 9.0
_BIG = 1e6

_ROWS = _B * _A
_ROW_BLK = 2048
_NB = 8
_GATHER_W = 128


def _index_kernel(s_ref, f_ref, idx_ref, bad_ref, off_ref):
    s = s_ref[...]
    ms = jnp.max(s, axis=1, keepdims=True)
    ci = lax.broadcasted_iota(jnp.int32, s.shape, 1)
    mi = jnp.min(jnp.where(s == ms, ci, _C), axis=1, keepdims=True)
    bad = (ms < _AGENT_THRESH) | (mi > 4)

    f = f_ref[...]
    fi = lax.broadcasted_iota(jnp.int32, f.shape, 1)
    mf = jnp.max(f, axis=1, keepdims=True)
    best = jnp.min(jnp.where(f == mf, fi, _M), axis=1, keepdims=True)

    r = (pl.program_id(0) * _ROW_BLK
         + lax.broadcasted_iota(jnp.int32, (_ROW_BLK, 1), 0))
    k = r * _M + best
    s = (k * 3) // 2
    idx_ref[...] = jnp.concatenate([s, s + 1], axis=1)
    bad_ref[...] = jnp.where(bad, 1.0, 0.0)
    off_ref[...] = jnp.where(k % 2 == 1, 8.0, 0.0)


def _compute_indices(asp, afcp):
    asp2 = asp.reshape(_ROWS, _C)
    afcp2 = afcp.reshape(_ROWS, _M)
    grid = _ROWS // _ROW_BLK
    idx, badf, offf = pl.pallas_call(
        _index_kernel,
        grid=(grid,),
        in_specs=[
            pl.BlockSpec((_ROW_BLK, _C), lambda i: (i, 0)),
            pl.BlockSpec((_ROW_BLK, _M), lambda i: (i, 0)),
        ],
        out_specs=[
            pl.BlockSpec((_ROW_BLK, 2), lambda i: (i, 0)),
            pl.BlockSpec((_ROW_BLK, 1), lambda i: (i, 0)),
            pl.BlockSpec((_ROW_BLK, 1), lambda i: (i, 0)),
        ],
        out_shape=[
            jax.ShapeDtypeStruct((_ROWS, 2), jnp.int32),
            jax.ShapeDtypeStruct((_ROWS, 1), jnp.float32),
            jax.ShapeDtypeStruct((_ROWS, 1), jnp.float32),
        ],
    )(asp2, afcp2)
    return idx, badf, offf


_N16 = _ROWS * _M * _T2 // 16
_NIDX = 2 * _ROWS


def _sc_gather(afp_flat, idx):
    mesh = plsc.VectorSubcoreMesh(core_axis_name="core",
                                  subcore_axis_name="subcore")

    @pl.kernel(out_type=jax.ShapeDtypeStruct((_NIDX, 16), jnp.float32),
               mesh=mesh,
               compiler_params=pltpu.CompilerParams(use_tc_tiling_on_sc=False))
    def gather_kernel(x_hbm, i_hbm, o_hbm):
        def body(i_vmem, o_vmem):
            pltpu.sync_copy(x_hbm.at[i_vmem.at[0]], o_vmem)

        pltpu.emit_pipeline(
            body,
            grid=(_NIDX // _GATHER_W,),
            in_specs=[pl.BlockSpec((1, _GATHER_W), index_map=lambda i: (0, i))],
            out_specs=[pl.BlockSpec((_GATHER_W, 16),
                                    index_map=lambda i: (i, 0))],
            core_axis_name=("core", "subcore"),
            dimension_semantics=(pltpu.PARALLEL,),
        )(i_hbm, o_hbm)

    return gather_kernel(afp_flat, idx)


def _loss_kernel(g_ref, bad_ref, off_ref, ap_ref, ego_ref, out_ref):
    g32 = g_ref[...]
    off = off_ref[...]
    g = jnp.where(off > 4.0, g32[:, :, 8:8 + _T2], g32[:, :, 0:_T2])
    ap = ap_ref[...]
    ego = ego_ref[...]

    lane = lax.broadcasted_iota(jnp.int32, g.shape, 2)
    apx = ap[:, :, 0:1]
    apy = ap[:, :, 1:2]
    c = jnp.where(lane == 0, g + apx, jnp.where(lane == 1, g + apy, g))
    for s in (2, 4, 8, 16):
        c = c + jnp.where(lane >= s, jnp.roll(c, s, axis=2), 0.0)

    lane2 = lax.broadcasted_iota(jnp.int32, ego.shape, 1)
    p = ego
    for s in (2, 4, 8, 16):
        p = p + jnp.where(lane2 >= s, jnp.roll(p, s, axis=1), 0.0)
    pred = p[:, None, :]

    d = pred - c
    s2 = d * d
    pair = s2 + jnp.where(lane % 2 == 0,
                          jnp.roll(s2, -1, axis=2),
                          jnp.roll(s2, 1, axis=2))
    bad = bad_ref[...]
    far = (pair > _DIS_TH_SQ) | (bad > 0.5)
    tm = jnp.where(far, _BIG, c)
    ad = jnp.abs(pred - tm)
    m = jnp.min(ad, axis=1)

    th = jnp.where(lane2 % 2 == 0, _X_TH, _Y_TH)
    contrib = jnp.where(m > th, 0.0, th - m)
    val = jnp.sum(contrib, keepdims=True) * (1.0 / (_B * _T2))

    @pl.when(pl.program_id(0) == 0)
    def _():
        out_ref[...] = jnp.zeros_like(out_ref)

    out_ref[...] += val


def _compute_loss(gathered, badf, offf, ap, ego):
    g3 = gathered.reshape(_B, _A, 32)
    bad3 = badf.reshape(_B, _A, 1)
    off3 = offf.reshape(_B, _A, 1)
    ego2 = ego.reshape(_B, _T2)
    grid = _B // _NB
    out = pl.pallas_call(
        _loss_kernel,
        grid=(grid,),
        in_specs=[
            pl.BlockSpec((_NB, _A, 32), lambda i: (i, 0, 0)),
            pl.BlockSpec((_NB, _A, 1), lambda i: (i, 0, 0)),
            pl.BlockSpec((_NB, _A, 1), lambda i: (i, 0, 0)),
            pl.BlockSpec((_NB, _A, 2), lambda i: (i, 0, 0)),
            pl.BlockSpec((_NB, _T2), lambda i: (i, 0)),
        ],
        out_specs=pl.BlockSpec((1, 1), lambda i: (0, 0)),
        out_shape=jax.ShapeDtypeStruct((1, 1), jnp.float32),
    )(g3, bad3, off3, ap, ego2)
    return out.reshape(())


def kernel(ego_fut_preds, agent_preds, agent_fut_preds, agent_score_preds,
           agent_fut_cls_preds):
    idx, badf, offf = _compute_indices(agent_score_preds, agent_fut_cls_preds)
    afp_flat = agent_fut_preds.reshape(_N16, 16)
    gathered = _sc_gather(afp_flat, idx.reshape(1, _NIDX))
    return _compute_loss(gathered, badf, offf, agent_preds, ego_fut_preds)


# speedup vs baseline: 173.2354x; 173.2354x over previous
import jax
import jax.numpy as jnp
from jax import lax
from jax.experimental import pallas as pl
from jax.experimental.pallas import tpu as pltpu

_B, _A, _M, _T, _C = 1024, 300, 6, 12, 10

_AGENT_THRESH = 0.5
_X_TH = 1.5
_Y_TH = 3.0
_DIS_TH_SQ = 9.0
_BIG = 1e6

_ABLK = 8
_GRID = (_A + _ABLK - 1) // _ABLK


def _loss_kernel(afp_ref, asp_ref, afcp_ref, ap_ref, ego_ref, out_ref,
                 min_ref):
    i = pl.program_id(0)

    @pl.when(i == 0)
    def _():
        min_ref[...] = jnp.full_like(min_ref, 1e30)

    asp = asp_ref[...]
    ms = jnp.max(asp, axis=0)
    mi = jnp.full(ms.shape, _C - 1, jnp.int32)
    for c in range(_C - 1, -1, -1):
        mi = jnp.where(asp[c] == ms, c, mi)
    bad = (ms < _AGENT_THRESH) | (mi > 4)

    f = afcp_ref[...]
    mf = jnp.max(f, axis=0)
    best = jnp.full(mf.shape, _M - 1, jnp.int32)
    for m in range(_M - 1, -1, -1):
        best = jnp.where(f[m] == mf, m, best)

    afp = afp_ref[...].reshape(_ABLK, _M, _T, 2, afp_ref.shape[-1])
    sel = afp[:, 0]
    for m in range(1, _M):
        sel = jnp.where((best == m)[:, None, None, :], afp[:, m], sel)

    planes = [sel[:, 0]]
    for t in range(1, _T):
        planes.append(planes[-1] + sel[:, t])
    cum = jnp.stack(planes, axis=1)
    ap = ap_ref[...]
    target = cum + ap[:, None, :, :]

    ego = ego_ref[...]
    eplanes = [ego[0]]
    for t in range(1, _T):
        eplanes.append(eplanes[-1] + ego[t])
    pred = jnp.stack(eplanes, axis=0)

    d = pred[None] - target
    pair = jnp.sum(d * d, axis=2, keepdims=True)
    far = (pair > _DIS_TH_SQ) | bad[:, None, None, :]
    tm = jnp.where(far, _BIG, target)
    ad = jnp.abs(pred[None] - tm)

    a_glob = i * _ABLK + lax.broadcasted_iota(jnp.int32, ad.shape, 0)
    ad = jnp.where(a_glob < _A, ad, 1e30)

    min_ref[...] = jnp.minimum(min_ref[...], jnp.min(ad, axis=0))

    @pl.when(i == _GRID - 1)
    def _():
        mn = min_ref[...]
        dd = lax.broadcasted_iota(jnp.int32, mn.shape, 1)
        th = jnp.where(dd == 0, _X_TH, _Y_TH)
        contrib = jnp.where(mn > th, 0.0, th - mn)
        out_ref[...] = (jnp.sum(contrib) * (1.0 / (_B * _T * 2))).reshape(1, 1)


def kernel(ego_fut_preds, agent_preds, agent_fut_preds, agent_score_preds,
           agent_fut_cls_preds):
    b = ego_fut_preds.shape[0]
    asp_t = jnp.transpose(agent_score_preds, (2, 1, 0))
    afcp_t = jnp.transpose(agent_fut_cls_preds, (2, 1, 0))
    ap_t = jnp.transpose(agent_preds, (1, 2, 0))
    ego_t = jnp.transpose(ego_fut_preds, (1, 2, 0))
    afp_t = jnp.transpose(agent_fut_preds, (1, 2, 3, 4, 0)).reshape(
        _A * _M * _T, 2, b)

    out = pl.pallas_call(
        _loss_kernel,
        grid=(_GRID,),
        in_specs=[
            pl.BlockSpec((_ABLK * _M * _T, 2, b), lambda i: (i, 0, 0)),
            pl.BlockSpec((_C, _ABLK, b), lambda i: (0, i, 0)),
            pl.BlockSpec((_M, _ABLK, b), lambda i: (0, i, 0)),
            pl.BlockSpec((_ABLK, 2, b), lambda i: (i, 0, 0)),
            pl.BlockSpec((_T, 2, b), lambda i: (0, 0, 0)),
        ],
        out_specs=pl.BlockSpec((1, 1), lambda i: (0, 0)),
        out_shape=jax.ShapeDtypeStruct((1, 1), jnp.float32),
        scratch_shapes=[pltpu.VMEM((_T, 2, b), jnp.float32)],
    )(afp_t, asp_t, afcp_t, ap_t, ego_t)
    return out.reshape(())


# speedup vs baseline: 235.7596x; 1.3609x over previous
import jax
import jax.numpy as jnp
from jax import lax
from jax.experimental import pallas as pl
from jax.experimental.pallas import tpu as pltpu

_B, _A, _M, _T, _C = 1024, 300, 6, 12, 10

_AGENT_THRESH = 0.5
_X_TH = 1.5
_Y_TH = 3.0
_DIS_TH_SQ = 9.0
_BIG = 1e6

_ABLK = 8
_GRID = (_A + _ABLK - 1) // _ABLK


def _loss_kernel(afp_ref, asp_ref, afcp_ref, ap_ref, ego_ref, out_ref,
                 minx_ref, miny_ref):
    i = pl.program_id(0)

    @pl.when(i == 0)
    def _():
        minx_ref[...] = jnp.full_like(minx_ref, 1e30)
        miny_ref[...] = jnp.full_like(miny_ref, 1e30)

    asp = asp_ref[...]
    m1 = asp[0]
    for c in range(1, 5):
        m1 = jnp.maximum(m1, asp[c])
    m2 = asp[5]
    for c in range(6, _C):
        m2 = jnp.maximum(m2, asp[c])
    bad = (jnp.maximum(m1, m2) < _AGENT_THRESH) | (m1 < m2)

    f = afcp_ref[...]
    afp = afp_ref[...].reshape(_ABLK, _M, _T, 2, afp_ref.shape[-1])
    sel = afp[:, 0]
    runmax = f[0]
    for m in range(1, _M):
        upd = f[m] > runmax
        runmax = jnp.maximum(runmax, f[m])
        sel = jnp.where(upd[:, None, None, :], afp[:, m], sel)
    ap = ap_ref[...]
    planes = [sel[:, 0] + ap]
    for t in range(1, _T):
        planes.append(planes[-1] + sel[:, t])
    cum = jnp.stack(planes, axis=1)
    tx = cum[:, :, 0, :]
    ty = cum[:, :, 1, :]

    ego = ego_ref[...]
    eplanes = [ego[0]]
    for t in range(1, _T):
        eplanes.append(eplanes[-1] + ego[t])
    epred = jnp.stack(eplanes, axis=0)
    predx = epred[:, 0, :]
    predy = epred[:, 1, :]

    dx = predx[None] - tx
    dy = predy[None] - ty
    far = (dx * dx + dy * dy > _DIS_TH_SQ) | bad[:, None, :]
    adx = jnp.where(far, _BIG - predx[None], jnp.abs(dx))
    ady = jnp.where(far, _BIG - predy[None], jnp.abs(dy))

    a_glob = i * _ABLK + lax.broadcasted_iota(jnp.int32, adx.shape, 0)
    valid = a_glob < _A
    adx = jnp.where(valid, adx, 1e30)
    ady = jnp.where(valid, ady, 1e30)

    minx_ref[...] = jnp.minimum(minx_ref[...], jnp.min(adx, axis=0))
    miny_ref[...] = jnp.minimum(miny_ref[...], jnp.min(ady, axis=0))

    @pl.when(i == _GRID - 1)
    def _():
        mx = minx_ref[...]
        my = miny_ref[...]
        cx = jnp.where(mx > _X_TH, 0.0, _X_TH - mx)
        cy = jnp.where(my > _Y_TH, 0.0, _Y_TH - my)
        out_ref[...] = ((jnp.sum(cx) + jnp.sum(cy))
                        * (1.0 / (_B * _T * 2))).reshape(1, 1)


def kernel(ego_fut_preds, agent_preds, agent_fut_preds, agent_score_preds,
           agent_fut_cls_preds):
    b = ego_fut_preds.shape[0]
    asp_t = jnp.transpose(agent_score_preds, (2, 1, 0))
    afcp_t = jnp.transpose(agent_fut_cls_preds, (2, 1, 0))
    ap_t = jnp.transpose(agent_preds, (1, 2, 0))
    ego_t = jnp.transpose(ego_fut_preds, (1, 2, 0))
    afp_t = jnp.transpose(agent_fut_preds, (1, 2, 3, 4, 0)).reshape(
        _A * _M * _T, 2, b)

    out = pl.pallas_call(
        _loss_kernel,
        grid=(_GRID,),
        in_specs=[
            pl.BlockSpec((_ABLK * _M * _T, 2, b), lambda i: (i, 0, 0)),
            pl.BlockSpec((_C, _ABLK, b), lambda i: (0, i, 0)),
            pl.BlockSpec((_M, _ABLK, b), lambda i: (0, i, 0)),
            pl.BlockSpec((_ABLK, 2, b), lambda i: (i, 0, 0)),
            pl.BlockSpec((_T, 2, b), lambda i: (0, 0, 0)),
        ],
        out_specs=pl.BlockSpec((1, 1), lambda i: (0, 0)),
        out_shape=jax.ShapeDtypeStruct((1, 1), jnp.float32),
        scratch_shapes=[pltpu.VMEM((_T, b), jnp.float32),
                        pltpu.VMEM((_T, b), jnp.float32)],
    )(afp_t, asp_t, afcp_t, ap_t, ego_t)
    return out.reshape(())


# speedup vs baseline: 261.2583x; 1.1082x over previous
import jax
import jax.numpy as jnp
from jax import lax
from jax.experimental import pallas as pl
from jax.experimental.pallas import tpu as pltpu

_B, _A, _M, _T, _C = 1024, 300, 6, 12, 10

_AGENT_THRESH = 0.5
_X_TH = 1.5
_Y_TH = 3.0
_DIS_TH_SQ = 9.0
_BIG = 1e6

_ABLK = 8
_GRID = (_A + _ABLK - 1) // _ABLK


def _loss_kernel(afp_ref, asp_ref, afcp_ref, ap_ref, ego_ref, out_ref,
                 mn_ref):
    i = pl.program_id(0)
    b = afp_ref.shape[-1]

    @pl.when(i == 0)
    def _():
        mn_ref[...] = jnp.full_like(mn_ref, 1e30)

    asp = asp_ref[...]
    m1 = asp[0]
    for c in range(1, 5):
        m1 = jnp.maximum(m1, asp[c])
    m2 = asp[5]
    for c in range(6, _C):
        m2 = jnp.maximum(m2, asp[c])
    bad = (jnp.maximum(m1, m2) < _AGENT_THRESH) | (m1 < m2)
    bad2 = bad[:, None, :]

    f = afcp_ref[...]
    runmax = f[0]
    upd2 = []
    for m in range(1, _M):
        upd = f[m] > runmax
        runmax = jnp.maximum(runmax, f[m])
        upd2.append(upd[:, None, :])

    a_glob = i * _ABLK + lax.broadcasted_iota(jnp.int32, (_ABLK, 2, b), 0)
    valid = a_glob < _A

    ego = ego_ref[...]
    afp = afp_ref[...].reshape(_ABLK, _M, _T, 2, b)
    ap = ap_ref[...]

    cum = ap
    pred = jnp.zeros((2, b), jnp.float32)
    for t in range(_T):
        sel_t = afp[:, 0, t]
        for m in range(1, _M):
            sel_t = jnp.where(upd2[m - 1], afp[:, m, t], sel_t)
        cum = cum + sel_t
        pred = pred + ego[t]
        d = pred[None] - cum
        s2 = d * d
        pair = s2 + jnp.roll(s2, 1, axis=1)
        far = (pair > _DIS_TH_SQ) | bad2
        ad = jnp.where(far, _BIG - pred[None], jnp.abs(d))
        ad = jnp.where(valid, ad, 1e30)
        mn_ref[t] = jnp.minimum(mn_ref[t], jnp.min(ad, axis=0))

    @pl.when(i == _GRID - 1)
    def _():
        mn = mn_ref[...]
        dd = lax.broadcasted_iota(jnp.int32, mn.shape, 1)
        th = jnp.where(dd == 0, _X_TH, _Y_TH)
        contrib = jnp.where(mn > th, 0.0, th - mn)
        out_ref[...] = (jnp.sum(contrib) * (1.0 / (_B * _T * 2))).reshape(1, 1)


def kernel(ego_fut_preds, agent_preds, agent_fut_preds, agent_score_preds,
           agent_fut_cls_preds):
    b = ego_fut_preds.shape[0]
    asp_t = jnp.transpose(agent_score_preds, (2, 1, 0))
    afcp_t = jnp.transpose(agent_fut_cls_preds, (2, 1, 0))
    ap_t = jnp.transpose(agent_preds, (1, 2, 0))
    ego_t = jnp.transpose(ego_fut_preds, (1, 2, 0))
    afp_t = jnp.transpose(agent_fut_preds, (1, 2, 3, 4, 0)).reshape(
        _A * _M * _T, 2, b)

    out = pl.pallas_call(
        _loss_kernel,
        grid=(_GRID,),
        in_specs=[
            pl.BlockSpec((_ABLK * _M * _T, 2, b), lambda i: (i, 0, 0)),
            pl.BlockSpec((_C, _ABLK, b), lambda i: (0, i, 0)),
            pl.BlockSpec((_M, _ABLK, b), lambda i: (0, i, 0)),
            pl.BlockSpec((_ABLK, 2, b), lambda i: (i, 0, 0)),
            pl.BlockSpec((_T, 2, b), lambda i: (0, 0, 0)),
        ],
        out_specs=pl.BlockSpec((1, 1), lambda i: (0, 0)),
        out_shape=jax.ShapeDtypeStruct((1, 1), jnp.float32),
        scratch_shapes=[pltpu.VMEM((_T, 2, b), jnp.float32)],
    )(afp_t, asp_t, afcp_t, ap_t, ego_t)
    return out.reshape(())


# speedup vs baseline: 267.3985x; 1.0235x over previous
import jax
import jax.numpy as jnp
from jax import lax
from jax.experimental import pallas as pl
from jax.experimental.pallas import tpu as pltpu

_B, _A, _M, _T, _C = 1024, 300, 6, 12, 10

_AGENT_THRESH = 0.5
_X_TH = 1.5
_Y_TH = 3.0
_DIS_TH_SQ = 9.0
_BIG = 1e6

_ABLK = 8
_GRID = (_A + _ABLK - 1) // _ABLK


def _loss_kernel(afp_ref, asp_ref, afcp_ref, ap_ref, ego_ref, out_ref,
                 mn_ref):
    i = pl.program_id(0)
    b = afp_ref.shape[-1]

    @pl.when(i == 0)
    def _():
        mn_ref[...] = jnp.full_like(mn_ref, 1e30)

    asp = asp_ref[...]
    m1 = asp[0]
    for c in range(1, 5):
        m1 = jnp.maximum(m1, asp[c])
    m2 = asp[5]
    for c in range(6, _C):
        m2 = jnp.maximum(m2, asp[c])
    bad = (jnp.maximum(m1, m2) < _AGENT_THRESH) | (m1 < m2)

    f = afcp_ref[...]
    runmax = f[0]
    best = jnp.zeros(runmax.shape, jnp.int32)
    for m in range(1, _M):
        upd = f[m] > runmax
        runmax = jnp.maximum(runmax, f[m])
        best = jnp.where(upd, m, best)

    a_sub = i * _ABLK + lax.broadcasted_iota(jnp.int32, bad.shape, 0)
    code = best + jnp.where(bad | (a_sub >= _A), _M + 2, 0)
    code2 = code[:, None, :]
    bad2 = code2 >= _M + 2

    ego = ego_ref[...]
    afp = afp_ref[...].reshape(_ABLK, _M, _T, 2, b)
    ap = ap_ref[...]

    cum = ap
    pred = jnp.zeros((2, b), jnp.float32)
    for t in range(_T):
        sel_t = afp[:, 0, t]
        for m in range(1, _M):
            sel_t = jnp.where(code2 == m, afp[:, m, t], sel_t)
        cum = cum + sel_t
        pred = pred + ego[t]
        d = pred[None] - cum
        s2 = d * d
        pair = s2 + jnp.roll(s2, 1, axis=1)
        far = (pair > _DIS_TH_SQ) | bad2
        ad = jnp.where(far, _BIG - pred[None], jnp.abs(d))
        mn_ref[t] = jnp.minimum(mn_ref[t], jnp.min(ad, axis=0))

    @pl.when(i == _GRID - 1)
    def _():
        mn = mn_ref[...]
        dd = lax.broadcasted_iota(jnp.int32, mn.shape, 1)
        th = jnp.where(dd == 0, _X_TH, _Y_TH)
        contrib = jnp.where(mn > th, 0.0, th - mn)
        out_ref[...] = (jnp.sum(contrib) * (1.0 / (_B * _T * 2))).reshape(1, 1)


def kernel(ego_fut_preds, agent_preds, agent_fut_preds, agent_score_preds,
           agent_fut_cls_preds):
    b = ego_fut_preds.shape[0]
    asp_t = jnp.transpose(agent_score_preds, (2, 1, 0))
    afcp_t = jnp.transpose(agent_fut_cls_preds, (2, 1, 0))
    ap_t = jnp.transpose(agent_preds, (1, 2, 0))
    ego_t = jnp.transpose(ego_fut_preds, (1, 2, 0))
    afp_t = jnp.transpose(agent_fut_preds, (1, 2, 3, 4, 0)).reshape(
        _A * _M * _T, 2, b)

    out = pl.pallas_call(
        _loss_kernel,
        grid=(_GRID,),
        in_specs=[
            pl.BlockSpec((_ABLK * _M * _T, 2, b), lambda i: (i, 0, 0)),
            pl.BlockSpec((_C, _ABLK, b), lambda i: (0, i, 0)),
            pl.BlockSpec((_M, _ABLK, b), lambda i: (0, i, 0)),
            pl.BlockSpec((_ABLK, 2, b), lambda i: (i, 0, 0)),
            pl.BlockSpec((_T, 2, b), lambda i: (0, 0, 0)),
        ],
        out_specs=pl.BlockSpec((1, 1), lambda i: (0, 0)),
        out_shape=jax.ShapeDtypeStruct((1, 1), jnp.float32),
        scratch_shapes=[pltpu.VMEM((_T, 2, b), jnp.float32)],
    )(afp_t, asp_t, afcp_t, ap_t, ego_t)
    return out.reshape(())
